# Initial kernel scaffold; baseline (speedup 1.0000x reference)
import jax
import jax.numpy as jnp
from jax.experimental import pallas as pl


def kernel(positions, graph_labels, weight, node_attrs, batch, atomic_masses, w_q):
    raise NotImplementedError("write your pallas kernel here")


# speedup vs baseline: 8.1375x; 8.1375x over previous
import functools

import jax
import jax.numpy as jnp
from jax import lax
from jax.experimental import pallas as pl
from jax.experimental.pallas import tpu as pltpu
from jax.experimental.pallas import tpu_sc as plsc

N = 131072
G = 2048
NW = 32
C = N // NW
GAMMA = 10000.0


def _sc_segment_sums(pos_flat, attr_flat, batch, mass16, wq16):
    mesh = plsc.VectorSubcoreMesh(core_axis_name="c", subcore_axis_name="s")

    @functools.partial(
        pl.kernel,
        out_type=[
            jax.ShapeDtypeStruct((2, 16, 128), jnp.float32),
            jax.ShapeDtypeStruct((2, 16, 128), jnp.float32),
        ],
        mesh=mesh,
        compiler_params=pltpu.CompilerParams(needs_layout_passes=False),
        scratch_types=[
            pltpu.VMEM((C * 3,), jnp.float32),
            pltpu.VMEM((C * 8,), jnp.float32),
            pltpu.VMEM((C + 16,), jnp.int32),
            pltpu.VMEM((16,), jnp.float32),
            pltpu.VMEM((16,), jnp.float32),
            pltpu.VMEM((16, 128), jnp.float32),
            pltpu.VMEM((16, 128), jnp.float32),
            pltpu.VMEM((16,), jnp.int32),
            pltpu.VMEM_SHARED((16, 128), jnp.float32),
            pltpu.VMEM_SHARED((16, 128), jnp.float32),
        ],
    )
    def k(pos_hbm, attr_hbm, batch_hbm, mass_hbm, wq_hbm, out_s, out_r,
          pos_v, attr_v, batch_v, mass_v, wq_v, acc_s, acc_r, rows_v, sh_s, sh_r):
        cid = lax.axis_index("c")
        sid = lax.axis_index("s")
        wid = sid * 2 + cid
        base = wid * C

        pltpu.sync_copy(pos_hbm.at[pl.ds(base * 3, C * 3)], pos_v)
        pltpu.sync_copy(attr_hbm.at[pl.ds(base * 8, C * 8)], attr_v)
        pltpu.sync_copy(batch_hbm.at[pl.ds(base, C)], batch_v.at[pl.ds(0, C)])
        pltpu.sync_copy(mass_hbm, mass_v)
        pltpu.sync_copy(wq_hbm, wq_v)

        lane = lax.iota(jnp.int32, 16)
        zeros16 = jnp.zeros((16,), jnp.float32)
        for rr in range(16):
            for cc in range(8):
                acc_s[rr, pl.ds(cc * 16, 16)] = zeros16
                acc_r[rr, pl.ds(cc * 16, 16)] = zeros16
        rows_v[...] = lane

        batch_v[pl.ds(C, 16)] = plsc.load_gather(
            batch_v, [jnp.full((16,), C - 1, jnp.int32)])

        wqv = wq_v[...]
        imv = 1.0 / mass_v[...]
        wq0 = jnp.sum(jnp.where(lane == 0, wqv, 0.0))
        wq1 = jnp.sum(jnp.where(lane == 1, wqv, 0.0))
        wq2 = jnp.sum(jnp.where(lane == 2, wqv, 0.0))
        im = [jnp.sum(jnp.where(lane == t, imv, 0.0)) for t in range(8)]
        is15 = lane == 15
        iota3 = lane * 3
        iota8 = lane * 8

        def body(j, carry):
            b16 = j * 16
            idx = batch_v[pl.ds(b16, 16)]
            nxt = plsc.load_gather(batch_v, [lane + (b16 + 1)])
            p3 = b16 * 3
            x = plsc.load_gather(pos_v, [iota3 + p3])
            y = plsc.load_gather(pos_v, [iota3 + (p3 + 1)])
            z = plsc.load_gather(pos_v, [iota3 + (p3 + 2)])
            v = x * wq0 + y * wq1 + z * wq2
            p8 = b16 * 8
            u = plsc.load_gather(attr_v, [iota8 + p8]) * im[0]
            for t in range(1, 8):
                u = u + plsc.load_gather(attr_v, [iota8 + (p8 + t)]) * im[t]
            cv = plsc.cumsum(v)
            cu = plsc.cumsum(u)
            bnd = idx != nxt
            mend = bnd | is15
            mneg = bnd & jnp.logical_not(is15)
            ri = idx >> 7
            ci = idx & 127
            rn = nxt >> 7
            cn = nxt & 127
            plsc.addupdate_scatter(acc_s, [ri, ci], cv, mask=mend)
            plsc.addupdate_scatter(acc_s, [rn, cn], -cv, mask=mneg)
            plsc.addupdate_scatter(acc_r, [ri, ci], cu, mask=mend)
            plsc.addupdate_scatter(acc_r, [rn, cn], -cu, mask=mneg)
            return carry

        lax.fori_loop(0, C // 16, body, 0)

        @pl.when(sid == 0)
        def _():
            pltpu.sync_copy(acc_s, sh_s)
            pltpu.sync_copy(acc_r, sh_r)

        plsc.subcore_barrier()

        @pl.when(sid != 0)
        def _():
            pltpu.sync_copy(acc_s, sh_s.at[rows_v], add=True)
            pltpu.sync_copy(acc_r, sh_r.at[rows_v], add=True)

        plsc.subcore_barrier()

        @pl.when(sid == 0)
        def _():
            pltpu.sync_copy(sh_s, out_s.at[cid])
            pltpu.sync_copy(sh_r, out_r.at[cid])

    return k(pos_flat, attr_flat, batch, mass16, wq16)


def _tc_final(s_ref, r_ref, w_ref, l_ref, wq_ref, o_loss, o_lv, o_la, o_lb):
    s = s_ref[0] + s_ref[1]
    r = r_ref[0] + r_ref[1]
    wq2 = jnp.sum(wq_ref[...] ** 2)
    q = 1.0 / (1.0 + jnp.exp(-s))
    dq = q * (1.0 - q)
    vals = dq * dq * wq2 * r * w_ref[...]
    lab = l_ref[...]
    mt = lab > 1
    ma = lab == 0
    mb = lab == 1
    nt = jnp.maximum(jnp.sum(mt.astype(jnp.float32)), 1.0)
    na = jnp.maximum(jnp.sum(ma.astype(jnp.float32)), 1.0)
    nb = jnp.maximum(jnp.sum(mb.astype(jnp.float32)), 1.0)
    lv = jnp.sum(jnp.where(mt, vals, 0.0)) / nt
    la = jnp.sum(jnp.where(ma, q * q, 0.0)) / na
    lb = jnp.sum(jnp.where(mb, (q - 1.0) ** 2, 0.0)) / nb
    lgv = jnp.log(lv)
    o_loss[0, 0] = lgv + GAMMA * (la + lb)
    o_lv[0, 0] = lgv
    o_la[0, 0] = GAMMA * la
    o_lb[0, 0] = GAMMA * lb


def kernel(positions, graph_labels, weight, node_attrs, batch, atomic_masses, w_q):
    pos_flat = positions.reshape(-1)
    attr_flat = node_attrs.reshape(-1)
    batch_i = batch.astype(jnp.int32)
    mass16 = jnp.concatenate([atomic_masses.astype(jnp.float32),
                              jnp.ones((8,), jnp.float32)])
    wq16 = jnp.pad(w_q.astype(jnp.float32), (0, 13))

    s_part, r_part = _sc_segment_sums(pos_flat, attr_flat, batch_i, mass16, wq16)

    wgt2d = weight.astype(jnp.float32).reshape(16, 128)
    lab2d = graph_labels.astype(jnp.int32).reshape(16, 128)
    wq128 = jnp.pad(w_q.astype(jnp.float32), (0, 125)).reshape(1, 128)

    scalar = jax.ShapeDtypeStruct((1, 1), jnp.float32)
    outs = pl.pallas_call(
        _tc_final,
        out_shape=[scalar, scalar, scalar, scalar],
        out_specs=[pl.BlockSpec(memory_space=pltpu.SMEM)] * 4,
    )(s_part, r_part, wgt2d, lab2d, wq128)
    loss, lgv, la, lb = (o.reshape(()) for o in outs)
    return (loss, lgv, la, lb)


# speedup vs baseline: 9.9020x; 1.2168x over previous
import functools

import jax
import jax.numpy as jnp
from jax import lax
from jax.experimental import pallas as pl
from jax.experimental.pallas import tpu as pltpu
from jax.experimental.pallas import tpu_sc as plsc

N = 131072
G = 2048
NW = 32
C = N // NW
BN = 2048
GAMMA = 10000.0


def _tc_prep(pos_ref, attr_ref, wq_ref, mass_ref, v_ref, u_ref):
    dn = (((1,), (1,)), ((), ()))
    wq3 = wq_ref[:, :3]
    im8 = 1.0 / mass_ref[:, :8]
    v_ref[...] = jax.lax.dot_general(
        wq3, pos_ref[...], dn, preferred_element_type=jnp.float32).reshape(BN)
    u_ref[...] = jax.lax.dot_general(
        im8, attr_ref[...], dn, preferred_element_type=jnp.float32).reshape(BN)


def _sc_segment_sums(v, u, batch):
    mesh = plsc.VectorSubcoreMesh(core_axis_name="c", subcore_axis_name="s")

    @functools.partial(
        pl.kernel,
        out_type=[
            jax.ShapeDtypeStruct((2, 16, 128), jnp.float32),
            jax.ShapeDtypeStruct((2, 16, 128), jnp.float32),
        ],
        mesh=mesh,
        compiler_params=pltpu.CompilerParams(needs_layout_passes=False),
        scratch_types=[
            pltpu.VMEM((C,), jnp.float32),
            pltpu.VMEM((C,), jnp.float32),
            pltpu.VMEM((C + 16,), jnp.int32),
            pltpu.VMEM((16, 128), jnp.float32),
            pltpu.VMEM((16, 128), jnp.float32),
            pltpu.VMEM((16,), jnp.int32),
            pltpu.VMEM_SHARED((16, 128), jnp.float32),
            pltpu.VMEM_SHARED((16, 128), jnp.float32),
        ],
    )
    def k(v_hbm, u_hbm, batch_hbm, out_s, out_r,
          v_v, u_v, batch_v, acc_s, acc_r, rows_v, sh_s, sh_r):
        cid = lax.axis_index("c")
        sid = lax.axis_index("s")
        wid = sid * 2 + cid
        base = wid * C

        pltpu.sync_copy(v_hbm.at[pl.ds(base, C)], v_v)
        pltpu.sync_copy(u_hbm.at[pl.ds(base, C)], u_v)
        pltpu.sync_copy(batch_hbm.at[pl.ds(base, C)], batch_v.at[pl.ds(0, C)])

        lane = lax.iota(jnp.int32, 16)
        zeros16 = jnp.zeros((16,), jnp.float32)
        for rr in range(16):
            for cc in range(8):
                acc_s[rr, pl.ds(cc * 16, 16)] = zeros16
                acc_r[rr, pl.ds(cc * 16, 16)] = zeros16
        rows_v[...] = lane

        batch_v[pl.ds(C, 16)] = plsc.load_gather(
            batch_v, [jnp.full((16,), C - 1, jnp.int32)])

        is15 = lane == 15

        def body(j, carry):
            b16 = j * 16
            idx = batch_v[pl.ds(b16, 16)]
            nxt = plsc.load_gather(batch_v, [lane + (b16 + 1)])
            cv = plsc.cumsum(v_v[pl.ds(b16, 16)])
            cu = plsc.cumsum(u_v[pl.ds(b16, 16)])
            bnd = idx != nxt
            mend = bnd | is15
            mneg = bnd & jnp.logical_not(is15)
            ri = idx >> 7
            ci = idx & 127
            rn = nxt >> 7
            cn = nxt & 127
            plsc.addupdate_scatter(acc_s, [ri, ci], cv, mask=mend)
            plsc.addupdate_scatter(acc_s, [rn, cn], -cv, mask=mneg)
            plsc.addupdate_scatter(acc_r, [ri, ci], cu, mask=mend)
            plsc.addupdate_scatter(acc_r, [rn, cn], -cu, mask=mneg)
            return carry

        lax.fori_loop(0, C // 16, body, 0)

        @pl.when(sid == 0)
        def _():
            pltpu.sync_copy(acc_s, sh_s)
            pltpu.sync_copy(acc_r, sh_r)

        plsc.subcore_barrier()

        @pl.when(sid != 0)
        def _():
            pltpu.sync_copy(acc_s, sh_s.at[rows_v], add=True)
            pltpu.sync_copy(acc_r, sh_r.at[rows_v], add=True)

        plsc.subcore_barrier()

        @pl.when(sid == 0)
        def _():
            pltpu.sync_copy(sh_s, out_s.at[cid])
            pltpu.sync_copy(sh_r, out_r.at[cid])

    return k(v, u, batch)


def _tc_final(s_ref, r_ref, w_ref, l_ref, wq_ref, o_loss, o_lv, o_la, o_lb):
    s = s_ref[0] + s_ref[1]
    r = r_ref[0] + r_ref[1]
    wq2 = jnp.sum(wq_ref[...] ** 2)
    q = 1.0 / (1.0 + jnp.exp(-s))
    dq = q * (1.0 - q)
    vals = dq * dq * wq2 * r * w_ref[...]
    lab = l_ref[...]
    mt = lab > 1
    ma = lab == 0
    mb = lab == 1
    nt = jnp.maximum(jnp.sum(mt.astype(jnp.float32)), 1.0)
    na = jnp.maximum(jnp.sum(ma.astype(jnp.float32)), 1.0)
    nb = jnp.maximum(jnp.sum(mb.astype(jnp.float32)), 1.0)
    lv = jnp.sum(jnp.where(mt, vals, 0.0)) / nt
    la = jnp.sum(jnp.where(ma, q * q, 0.0)) / na
    lb = jnp.sum(jnp.where(mb, (q - 1.0) ** 2, 0.0)) / nb
    lgv = jnp.log(lv)
    o_loss[0, 0] = lgv + GAMMA * (la + lb)
    o_lv[0, 0] = lgv
    o_la[0, 0] = GAMMA * la
    o_lb[0, 0] = GAMMA * lb


def kernel(positions, graph_labels, weight, node_attrs, batch, atomic_masses, w_q):
    batch_i = batch.astype(jnp.int32)
    mass128 = jnp.pad(atomic_masses.astype(jnp.float32), (0, 120),
                      constant_values=1.0).reshape(1, 128)
    wq128 = jnp.pad(w_q.astype(jnp.float32), (0, 125)).reshape(1, 128)

    nodes = jax.ShapeDtypeStruct((N,), jnp.float32)
    v, u = pl.pallas_call(
        _tc_prep,
        grid=(N // BN,),
        in_specs=[
            pl.BlockSpec((BN, 3), lambda g: (g, 0)),
            pl.BlockSpec((BN, 8), lambda g: (g, 0)),
            pl.BlockSpec((1, 128), lambda g: (0, 0)),
            pl.BlockSpec((1, 128), lambda g: (0, 0)),
        ],
        out_specs=[
            pl.BlockSpec((BN,), lambda g: (g,)),
            pl.BlockSpec((BN,), lambda g: (g,)),
        ],
        out_shape=[nodes, nodes],
    )(positions, node_attrs, wq128, mass128)

    s_part, r_part = _sc_segment_sums(v, u, batch_i)

    wgt2d = weight.astype(jnp.float32).reshape(16, 128)
    lab2d = graph_labels.astype(jnp.int32).reshape(16, 128)

    scalar = jax.ShapeDtypeStruct((1, 1), jnp.float32)
    outs = pl.pallas_call(
        _tc_final,
        out_shape=[scalar, scalar, scalar, scalar],
        out_specs=[pl.BlockSpec(memory_space=pltpu.SMEM)] * 4,
    )(s_part, r_part, wgt2d, lab2d, wq128)
    loss, lgv, la, lb = (o.reshape(()) for o in outs)
    return (loss, lgv, la, lb)


# speedup vs baseline: 39.5541x; 3.9945x over previous
import functools

import jax
import jax.numpy as jnp
from jax import lax
from jax.experimental import pallas as pl
from jax.experimental.pallas import tpu as pltpu
from jax.experimental.pallas import tpu_sc as plsc

N = 131072
G = 2048
NW = 32
C = N // NW
BN = 8192
GAMMA = 10000.0


def _tc_prep(pos_ref, attr_ref, wq_ref, mass_ref, v_ref, u_ref):
    dn = (((1,), (0,)), ((), ()))
    wq3 = wq_ref[:, :3]
    im8 = 1.0 / mass_ref[:, :8]
    v_ref[...] = jax.lax.dot_general(
        wq3, pos_ref[...], dn, preferred_element_type=jnp.float32).reshape(BN)
    u_ref[...] = jax.lax.dot_general(
        im8, attr_ref[...], dn, preferred_element_type=jnp.float32).reshape(BN)


def _sc_segment_sums(v, u, batch):
    mesh = plsc.VectorSubcoreMesh(core_axis_name="c", subcore_axis_name="s")

    @functools.partial(
        pl.kernel,
        out_type=[
            jax.ShapeDtypeStruct((2, 16, 128), jnp.float32),
            jax.ShapeDtypeStruct((2, 16, 128), jnp.float32),
        ],
        mesh=mesh,
        compiler_params=pltpu.CompilerParams(needs_layout_passes=False),
        scratch_types=[
            pltpu.VMEM((C,), jnp.float32),
            pltpu.VMEM((C,), jnp.float32),
            pltpu.VMEM((C + 16,), jnp.int32),
            pltpu.VMEM((16, 128), jnp.float32),
            pltpu.VMEM((16, 128), jnp.float32),
            pltpu.VMEM((16,), jnp.int32),
            pltpu.VMEM_SHARED((16, 128), jnp.float32),
            pltpu.VMEM_SHARED((16, 128), jnp.float32),
        ],
    )
    def k(v_hbm, u_hbm, batch_hbm, out_s, out_r,
          v_v, u_v, batch_v, acc_s, acc_r, rows_v, sh_s, sh_r):
        cid = lax.axis_index("c")
        sid = lax.axis_index("s")
        wid = sid * 2 + cid
        base = wid * C

        pltpu.sync_copy(v_hbm.at[pl.ds(base, C)], v_v)
        pltpu.sync_copy(u_hbm.at[pl.ds(base, C)], u_v)
        pltpu.sync_copy(batch_hbm.at[pl.ds(base, C)], batch_v.at[pl.ds(0, C)])

        lane = lax.iota(jnp.int32, 16)
        zeros16 = jnp.zeros((16,), jnp.float32)
        for rr in range(16):
            for cc in range(8):
                acc_s[rr, pl.ds(cc * 16, 16)] = zeros16
                acc_r[rr, pl.ds(cc * 16, 16)] = zeros16
        rows_v[...] = lane

        batch_v[pl.ds(C, 16)] = plsc.load_gather(
            batch_v, [jnp.full((16,), C - 1, jnp.int32)])

        is15 = lane == 15

        def body(j, carry):
            b16 = j * 16
            idx = batch_v[pl.ds(b16, 16)]
            nxt = plsc.load_gather(batch_v, [lane + (b16 + 1)])
            cv = plsc.cumsum(v_v[pl.ds(b16, 16)])
            cu = plsc.cumsum(u_v[pl.ds(b16, 16)])
            bnd = idx != nxt
            mend = bnd | is15
            mneg = bnd & jnp.logical_not(is15)
            ri = idx >> 7
            ci = idx & 127
            rn = nxt >> 7
            cn = nxt & 127
            plsc.addupdate_scatter(acc_s, [ri, ci], cv, mask=mend)
            plsc.addupdate_scatter(acc_s, [rn, cn], -cv, mask=mneg)
            plsc.addupdate_scatter(acc_r, [ri, ci], cu, mask=mend)
            plsc.addupdate_scatter(acc_r, [rn, cn], -cu, mask=mneg)
            return carry

        lax.fori_loop(0, C // 16, body, 0)

        @pl.when(sid == 0)
        def _():
            pltpu.sync_copy(acc_s, sh_s)
            pltpu.sync_copy(acc_r, sh_r)

        plsc.subcore_barrier()

        @pl.when(sid != 0)
        def _():
            pltpu.sync_copy(acc_s, sh_s.at[rows_v], add=True)
            pltpu.sync_copy(acc_r, sh_r.at[rows_v], add=True)

        plsc.subcore_barrier()

        @pl.when(sid == 0)
        def _():
            pltpu.sync_copy(sh_s, out_s.at[cid])
            pltpu.sync_copy(sh_r, out_r.at[cid])

    return k(v, u, batch)


def _tc_final(s_ref, r_ref, w_ref, l_ref, wq_ref, o_loss, o_lv, o_la, o_lb):
    s = s_ref[0] + s_ref[1]
    r = r_ref[0] + r_ref[1]
    wq2 = jnp.sum(wq_ref[...] ** 2)
    q = 1.0 / (1.0 + jnp.exp(-s))
    dq = q * (1.0 - q)
    vals = dq * dq * wq2 * r * w_ref[...]
    lab = l_ref[...]
    mt = lab > 1
    ma = lab == 0
    mb = lab == 1
    nt = jnp.maximum(jnp.sum(mt.astype(jnp.float32)), 1.0)
    na = jnp.maximum(jnp.sum(ma.astype(jnp.float32)), 1.0)
    nb = jnp.maximum(jnp.sum(mb.astype(jnp.float32)), 1.0)
    lv = jnp.sum(jnp.where(mt, vals, 0.0)) / nt
    la = jnp.sum(jnp.where(ma, q * q, 0.0)) / na
    lb = jnp.sum(jnp.where(mb, (q - 1.0) ** 2, 0.0)) / nb
    lgv = jnp.log(lv)
    o_loss[0, 0] = lgv + GAMMA * (la + lb)
    o_lv[0, 0] = lgv
    o_la[0, 0] = GAMMA * la
    o_lb[0, 0] = GAMMA * lb


def kernel(positions, graph_labels, weight, node_attrs, batch, atomic_masses, w_q):
    batch_i = batch.astype(jnp.int32)
    mass128 = jnp.pad(atomic_masses.astype(jnp.float32), (0, 120),
                      constant_values=1.0).reshape(1, 128)
    wq128 = jnp.pad(w_q.astype(jnp.float32), (0, 125)).reshape(1, 128)

    nodes = jax.ShapeDtypeStruct((N,), jnp.float32)
    v, u = pl.pallas_call(
        _tc_prep,
        grid=(N // BN,),
        in_specs=[
            pl.BlockSpec((3, BN), lambda g: (0, g)),
            pl.BlockSpec((8, BN), lambda g: (0, g)),
            pl.BlockSpec((1, 128), lambda g: (0, 0)),
            pl.BlockSpec((1, 128), lambda g: (0, 0)),
        ],
        out_specs=[
            pl.BlockSpec((BN,), lambda g: (g,)),
            pl.BlockSpec((BN,), lambda g: (g,)),
        ],
        out_shape=[nodes, nodes],
    )(positions.T, node_attrs.T, wq128, mass128)

    s_part, r_part = _sc_segment_sums(v, u, batch_i)

    wgt2d = weight.astype(jnp.float32).reshape(16, 128)
    lab2d = graph_labels.astype(jnp.int32).reshape(16, 128)

    scalar = jax.ShapeDtypeStruct((1, 1), jnp.float32)
    outs = pl.pallas_call(
        _tc_final,
        out_shape=[scalar, scalar, scalar, scalar],
        out_specs=[pl.BlockSpec(memory_space=pltpu.SMEM)] * 4,
    )(s_part, r_part, wgt2d, lab2d, wq128)
    loss, lgv, la, lb = (o.reshape(()) for o in outs)
    return (loss, lgv, la, lb)


# speedup vs baseline: 43.7239x; 1.1054x over previous
import functools

import jax
import jax.numpy as jnp
from jax import lax
from jax.experimental import pallas as pl
from jax.experimental.pallas import tpu as pltpu
from jax.experimental.pallas import tpu_sc as plsc

N = 131072
G = 2048
NW = 32
C = N // NW
BN = 16384
GAMMA = 10000.0


def _tc_prep(pos_ref, attr_ref, wq_ref, mass_ref, v_ref, u_ref):
    dn = (((1,), (0,)), ((), ()))
    wq3 = wq_ref[:, :3]
    im8 = 1.0 / mass_ref[:, :8]
    v_ref[...] = jax.lax.dot_general(
        wq3, pos_ref[...], dn, preferred_element_type=jnp.float32).reshape(BN)
    u_ref[...] = jax.lax.dot_general(
        im8, attr_ref[...], dn, preferred_element_type=jnp.float32).reshape(BN)


def _sc_segment_sums(v, u, batch):
    mesh = plsc.VectorSubcoreMesh(core_axis_name="c", subcore_axis_name="s")

    @functools.partial(
        pl.kernel,
        out_type=[
            jax.ShapeDtypeStruct((2, 16, 128), jnp.float32),
            jax.ShapeDtypeStruct((2, 16, 128), jnp.float32),
        ],
        mesh=mesh,
        compiler_params=pltpu.CompilerParams(needs_layout_passes=False),
        scratch_types=[
            pltpu.VMEM((C,), jnp.float32),
            pltpu.VMEM((C,), jnp.float32),
            pltpu.VMEM((C + 16,), jnp.int32),
            pltpu.VMEM((16, 128), jnp.float32),
            pltpu.VMEM((16, 128), jnp.float32),
            pltpu.VMEM((16,), jnp.int32),
            pltpu.VMEM_SHARED((16, 128), jnp.float32),
            pltpu.VMEM_SHARED((16, 128), jnp.float32),
        ],
    )
    def k(v_hbm, u_hbm, batch_hbm, out_s, out_r,
          v_v, u_v, batch_v, acc_s, acc_r, rows_v, sh_s, sh_r):
        cid = lax.axis_index("c")
        sid = lax.axis_index("s")
        wid = sid * 2 + cid
        base = wid * C

        pltpu.sync_copy(v_hbm.at[pl.ds(base, C)], v_v)
        pltpu.sync_copy(u_hbm.at[pl.ds(base, C)], u_v)
        pltpu.sync_copy(batch_hbm.at[pl.ds(base, C)], batch_v.at[pl.ds(0, C)])

        lane = lax.iota(jnp.int32, 16)
        zeros16 = jnp.zeros((16,), jnp.float32)
        for rr in range(16):
            for cc in range(8):
                acc_s[rr, pl.ds(cc * 16, 16)] = zeros16
                acc_r[rr, pl.ds(cc * 16, 16)] = zeros16
        rows_v[...] = lane

        batch_v[pl.ds(C, 16)] = plsc.load_gather(
            batch_v, [jnp.full((16,), C - 1, jnp.int32)])

        is15 = lane == 15

        def group(b16):
            idx = batch_v[pl.ds(b16, 16)]
            nxt = plsc.load_gather(batch_v, [lane + (b16 + 1)])
            cv = plsc.cumsum(v_v[pl.ds(b16, 16)])
            cu = plsc.cumsum(u_v[pl.ds(b16, 16)])
            bnd = idx != nxt
            mend = bnd | is15
            mneg = bnd & jnp.logical_not(is15)
            ri = idx >> 7
            ci = idx & 127
            rn = nxt >> 7
            cn = nxt & 127
            plsc.addupdate_scatter(acc_s, [ri, ci], cv, mask=mend)
            plsc.addupdate_scatter(acc_s, [rn, cn], -cv, mask=mneg)
            plsc.addupdate_scatter(acc_r, [ri, ci], cu, mask=mend)
            plsc.addupdate_scatter(acc_r, [rn, cn], -cu, mask=mneg)

        UNROLL = 4

        def body(j, carry):
            for t in range(UNROLL):
                group(j * (16 * UNROLL) + 16 * t)
            return carry

        lax.fori_loop(0, C // (16 * UNROLL), body, 0)

        @pl.when(sid == 0)
        def _():
            pltpu.sync_copy(acc_s, sh_s)
            pltpu.sync_copy(acc_r, sh_r)

        plsc.subcore_barrier()

        @pl.when(sid != 0)
        def _():
            pltpu.sync_copy(acc_s, sh_s.at[rows_v], add=True)
            pltpu.sync_copy(acc_r, sh_r.at[rows_v], add=True)

        plsc.subcore_barrier()

        @pl.when(sid == 0)
        def _():
            pltpu.sync_copy(sh_s, out_s.at[cid])
            pltpu.sync_copy(sh_r, out_r.at[cid])

    return k(v, u, batch)


def _tc_final(s_ref, r_ref, w_ref, l_ref, wq_ref, o_loss, o_lv, o_la, o_lb):
    s = s_ref[0] + s_ref[1]
    r = r_ref[0] + r_ref[1]
    wq2 = jnp.sum(wq_ref[...] ** 2)
    q = 1.0 / (1.0 + jnp.exp(-s))
    dq = q * (1.0 - q)
    vals = dq * dq * wq2 * r * w_ref[...]
    lab = l_ref[...]
    mt = lab > 1
    ma = lab == 0
    mb = lab == 1
    nt = jnp.maximum(jnp.sum(mt.astype(jnp.float32)), 1.0)
    na = jnp.maximum(jnp.sum(ma.astype(jnp.float32)), 1.0)
    nb = jnp.maximum(jnp.sum(mb.astype(jnp.float32)), 1.0)
    lv = jnp.sum(jnp.where(mt, vals, 0.0)) / nt
    la = jnp.sum(jnp.where(ma, q * q, 0.0)) / na
    lb = jnp.sum(jnp.where(mb, (q - 1.0) ** 2, 0.0)) / nb
    lgv = jnp.log(lv)
    o_loss[0, 0] = lgv + GAMMA * (la + lb)
    o_lv[0, 0] = lgv
    o_la[0, 0] = GAMMA * la
    o_lb[0, 0] = GAMMA * lb


def kernel(positions, graph_labels, weight, node_attrs, batch, atomic_masses, w_q):
    batch_i = batch.astype(jnp.int32)
    mass128 = jnp.pad(atomic_masses.astype(jnp.float32), (0, 120),
                      constant_values=1.0).reshape(1, 128)
    wq128 = jnp.pad(w_q.astype(jnp.float32), (0, 125)).reshape(1, 128)

    nodes = jax.ShapeDtypeStruct((N,), jnp.float32)
    v, u = pl.pallas_call(
        _tc_prep,
        grid=(N // BN,),
        in_specs=[
            pl.BlockSpec((3, BN), lambda g: (0, g)),
            pl.BlockSpec((8, BN), lambda g: (0, g)),
            pl.BlockSpec((1, 128), lambda g: (0, 0)),
            pl.BlockSpec((1, 128), lambda g: (0, 0)),
        ],
        out_specs=[
            pl.BlockSpec((BN,), lambda g: (g,)),
            pl.BlockSpec((BN,), lambda g: (g,)),
        ],
        out_shape=[nodes, nodes],
    )(positions.T, node_attrs.T, wq128, mass128)

    s_part, r_part = _sc_segment_sums(v, u, batch_i)

    wgt2d = weight.astype(jnp.float32).reshape(16, 128)
    lab2d = graph_labels.astype(jnp.int32).reshape(16, 128)

    scalar = jax.ShapeDtypeStruct((1, 1), jnp.float32)
    outs = pl.pallas_call(
        _tc_final,
        out_shape=[scalar, scalar, scalar, scalar],
        out_specs=[pl.BlockSpec(memory_space=pltpu.SMEM)] * 4,
    )(s_part, r_part, wgt2d, lab2d, wq128)
    loss, lgv, la, lb = (o.reshape(()) for o in outs)
    return (loss, lgv, la, lb)


# speedup vs baseline: 48.5712x; 1.1109x over previous
import functools

import jax
import jax.numpy as jnp
from jax import lax
from jax.experimental import pallas as pl
from jax.experimental.pallas import tpu as pltpu
from jax.experimental.pallas import tpu_sc as plsc

N = 131072
G = 2048
NW = 32
C = N // NW
BN = 16384
GAMMA = 10000.0


def _tc_prep(pos_ref, attr_ref, wq_ref, mass_ref, v_ref, u_ref):
    dn = (((1,), (0,)), ((), ()))
    wq3 = wq_ref[:, :3]
    im8 = 1.0 / mass_ref[:, :8]
    v_ref[...] = jax.lax.dot_general(
        wq3, pos_ref[...], dn, preferred_element_type=jnp.float32).reshape(BN)
    u_ref[...] = jax.lax.dot_general(
        im8, attr_ref[...], dn, preferred_element_type=jnp.float32).reshape(BN)


def _sc_segment_sums(v, u, batch):
    mesh = plsc.VectorSubcoreMesh(core_axis_name="c", subcore_axis_name="s")

    @functools.partial(
        pl.kernel,
        out_type=[
            jax.ShapeDtypeStruct((2, 16, 128), jnp.float32),
            jax.ShapeDtypeStruct((2, 16, 128), jnp.float32),
        ],
        mesh=mesh,
        compiler_params=pltpu.CompilerParams(needs_layout_passes=False),
        scratch_types=[
            pltpu.VMEM((C,), jnp.float32),
            pltpu.VMEM((C,), jnp.float32),
            pltpu.VMEM((C + 16,), jnp.int32),
            pltpu.VMEM((16, 128), jnp.float32),
            pltpu.VMEM((16, 128), jnp.float32),
            pltpu.VMEM((16,), jnp.int32),
            pltpu.VMEM_SHARED((16, 128), jnp.float32),
            pltpu.VMEM_SHARED((16, 128), jnp.float32),
            pltpu.SemaphoreType.DMA,
        ],
    )
    def k(v_hbm, u_hbm, batch_hbm, out_s, out_r,
          v_v, u_v, batch_v, acc_s, acc_r, rows_v, sh_s, sh_r, sem):
        cid = lax.axis_index("c")
        sid = lax.axis_index("s")
        wid = sid * 2 + cid
        base = wid * C

        cp1 = pltpu.async_copy(v_hbm.at[pl.ds(base, C)], v_v, sem)
        cp2 = pltpu.async_copy(u_hbm.at[pl.ds(base, C)], u_v, sem)
        cp3 = pltpu.async_copy(batch_hbm.at[pl.ds(base, C)], batch_v.at[pl.ds(0, C)], sem)

        lane = lax.iota(jnp.int32, 16)
        zeros16 = jnp.zeros((16,), jnp.float32)
        for rr in range(16):
            for cc in range(8):
                acc_s[rr, pl.ds(cc * 16, 16)] = zeros16
                acc_r[rr, pl.ds(cc * 16, 16)] = zeros16
        rows_v[...] = lane

        cp1.wait()
        cp2.wait()
        cp3.wait()

        batch_v[pl.ds(C, 16)] = plsc.load_gather(
            batch_v, [jnp.full((16,), C - 1, jnp.int32)])

        is15 = lane == 15

        def group(b16):
            idx = batch_v[pl.ds(b16, 16)]
            nxt = plsc.load_gather(batch_v, [lane + (b16 + 1)])
            cv = plsc.cumsum(v_v[pl.ds(b16, 16)])
            cu = plsc.cumsum(u_v[pl.ds(b16, 16)])
            bnd = idx != nxt
            mend = bnd | is15
            mneg = bnd & jnp.logical_not(is15)
            ri = idx >> 7
            ci = idx & 127
            rn = nxt >> 7
            cn = nxt & 127
            plsc.addupdate_scatter(acc_s, [ri, ci], cv, mask=mend)
            plsc.addupdate_scatter(acc_s, [rn, cn], -cv, mask=mneg)
            plsc.addupdate_scatter(acc_r, [ri, ci], cu, mask=mend)
            plsc.addupdate_scatter(acc_r, [rn, cn], -cu, mask=mneg)

        @plsc.parallel_loop(0, C // 16, step=1, unroll=4)
        def _loop(j):
            group(j * 16)

        @pl.when(sid == 0)
        def _():
            pltpu.sync_copy(acc_s, sh_s)
            pltpu.sync_copy(acc_r, sh_r)

        plsc.subcore_barrier()

        @pl.when(sid != 0)
        def _():
            pltpu.sync_copy(acc_s, sh_s.at[rows_v], add=True)
            pltpu.sync_copy(acc_r, sh_r.at[rows_v], add=True)

        plsc.subcore_barrier()

        @pl.when(sid == 0)
        def _():
            pltpu.sync_copy(sh_s, out_s.at[cid])
            pltpu.sync_copy(sh_r, out_r.at[cid])

    return k(v, u, batch)


def _tc_final(s_ref, r_ref, w_ref, l_ref, wq_ref, o_loss, o_lv, o_la, o_lb):
    s = s_ref[0] + s_ref[1]
    r = r_ref[0] + r_ref[1]
    wq2 = jnp.sum(wq_ref[...] ** 2)
    q = 1.0 / (1.0 + jnp.exp(-s))
    dq = q * (1.0 - q)
    vals = dq * dq * wq2 * r * w_ref[...]
    lab = l_ref[...]
    mt = lab > 1
    ma = lab == 0
    mb = lab == 1
    nt = jnp.maximum(jnp.sum(mt.astype(jnp.float32)), 1.0)
    na = jnp.maximum(jnp.sum(ma.astype(jnp.float32)), 1.0)
    nb = jnp.maximum(jnp.sum(mb.astype(jnp.float32)), 1.0)
    lv = jnp.sum(jnp.where(mt, vals, 0.0)) / nt
    la = jnp.sum(jnp.where(ma, q * q, 0.0)) / na
    lb = jnp.sum(jnp.where(mb, (q - 1.0) ** 2, 0.0)) / nb
    lgv = jnp.log(lv)
    o_loss[0, 0] = lgv + GAMMA * (la + lb)
    o_lv[0, 0] = lgv
    o_la[0, 0] = GAMMA * la
    o_lb[0, 0] = GAMMA * lb


def kernel(positions, graph_labels, weight, node_attrs, batch, atomic_masses, w_q):
    batch_i = batch.astype(jnp.int32)
    mass128 = jnp.pad(atomic_masses.astype(jnp.float32), (0, 120),
                      constant_values=1.0).reshape(1, 128)
    wq128 = jnp.pad(w_q.astype(jnp.float32), (0, 125)).reshape(1, 128)

    nodes = jax.ShapeDtypeStruct((N,), jnp.float32)
    v, u = pl.pallas_call(
        _tc_prep,
        grid=(N // BN,),
        in_specs=[
            pl.BlockSpec((3, BN), lambda g: (0, g)),
            pl.BlockSpec((8, BN), lambda g: (0, g)),
            pl.BlockSpec((1, 128), lambda g: (0, 0)),
            pl.BlockSpec((1, 128), lambda g: (0, 0)),
        ],
        out_specs=[
            pl.BlockSpec((BN,), lambda g: (g,)),
            pl.BlockSpec((BN,), lambda g: (g,)),
        ],
        out_shape=[nodes, nodes],
    )(positions.T, node_attrs.T, wq128, mass128)

    s_part, r_part = _sc_segment_sums(v, u, batch_i)

    wgt2d = weight.astype(jnp.float32).reshape(16, 128)
    lab2d = graph_labels.astype(jnp.int32).reshape(16, 128)

    scalar = jax.ShapeDtypeStruct((1, 1), jnp.float32)
    outs = pl.pallas_call(
        _tc_final,
        out_shape=[scalar, scalar, scalar, scalar],
        out_specs=[pl.BlockSpec(memory_space=pltpu.SMEM)] * 4,
    )(s_part, r_part, wgt2d, lab2d, wq128)
    loss, lgv, la, lb = (o.reshape(()) for o in outs)
    return (loss, lgv, la, lb)


# speedup vs baseline: 51.9465x; 1.0695x over previous
import functools

import jax
import jax.numpy as jnp
from jax import lax
from jax.experimental import pallas as pl
from jax.experimental.pallas import tpu as pltpu
from jax.experimental.pallas import tpu_sc as plsc

N = 131072
G = 2048
NW = 32
C = N // NW
BN = 32768
GAMMA = 10000.0


def _tc_prep(pos_ref, attr_ref, wq_ref, mass_ref, v_ref, u_ref):
    dn = (((1,), (0,)), ((), ()))
    wq3 = wq_ref[:, :3]
    im8 = 1.0 / mass_ref[:, :8]
    v_ref[...] = jax.lax.dot_general(
        wq3, pos_ref[...], dn, preferred_element_type=jnp.float32).reshape(BN)
    u_ref[...] = jax.lax.dot_general(
        im8, attr_ref[...], dn, preferred_element_type=jnp.float32).reshape(BN)


def _sc_segment_sums(v, u, batch):
    mesh = plsc.VectorSubcoreMesh(core_axis_name="c", subcore_axis_name="s")

    @functools.partial(
        pl.kernel,
        out_type=[
            jax.ShapeDtypeStruct((2, 16, 128), jnp.float32),
            jax.ShapeDtypeStruct((2, 16, 128), jnp.float32),
        ],
        mesh=mesh,
        compiler_params=pltpu.CompilerParams(needs_layout_passes=False),
        scratch_types=[
            pltpu.VMEM((C,), jnp.float32),
            pltpu.VMEM((C,), jnp.float32),
            pltpu.VMEM((C + 16,), jnp.int32),
            pltpu.VMEM((16, 128), jnp.float32),
            pltpu.VMEM((16, 128), jnp.float32),
            pltpu.VMEM((16,), jnp.int32),
            pltpu.VMEM_SHARED((16, 128), jnp.float32),
            pltpu.VMEM_SHARED((16, 128), jnp.float32),
            pltpu.SemaphoreType.DMA,
        ],
    )
    def k(v_hbm, u_hbm, batch_hbm, out_s, out_r,
          v_v, u_v, batch_v, acc_s, acc_r, rows_v, sh_s, sh_r, sem):
        cid = lax.axis_index("c")
        sid = lax.axis_index("s")
        wid = sid * 2 + cid
        base = wid * C

        cp1 = pltpu.async_copy(v_hbm.at[pl.ds(base, C)], v_v, sem)
        cp2 = pltpu.async_copy(u_hbm.at[pl.ds(base, C)], u_v, sem)
        cp3 = pltpu.async_copy(batch_hbm.at[pl.ds(base, C)], batch_v.at[pl.ds(0, C)], sem)

        lane = lax.iota(jnp.int32, 16)
        zeros16 = jnp.zeros((16,), jnp.float32)
        for rr in range(16):
            for cc in range(8):
                acc_s[rr, pl.ds(cc * 16, 16)] = zeros16
                acc_r[rr, pl.ds(cc * 16, 16)] = zeros16
        rows_v[...] = lane

        cp1.wait()
        cp2.wait()
        cp3.wait()

        batch_v[pl.ds(C, 16)] = plsc.load_gather(
            batch_v, [jnp.full((16,), C - 1, jnp.int32)])

        is15 = lane == 15

        def group(b16):
            idx = batch_v[pl.ds(b16, 16)]
            nxt = plsc.load_gather(batch_v, [lane + (b16 + 1)])
            cv = plsc.cumsum(v_v[pl.ds(b16, 16)])
            cu = plsc.cumsum(u_v[pl.ds(b16, 16)])
            bnd = idx != nxt
            mend = bnd | is15
            mneg = bnd & jnp.logical_not(is15)
            ri = idx >> 7
            ci = idx & 127
            rn = nxt >> 7
            cn = nxt & 127
            plsc.addupdate_scatter(acc_s, [ri, ci], cv, mask=mend)
            plsc.addupdate_scatter(acc_s, [rn, cn], -cv, mask=mneg)
            plsc.addupdate_scatter(acc_r, [ri, ci], cu, mask=mend)
            plsc.addupdate_scatter(acc_r, [rn, cn], -cu, mask=mneg)

        @plsc.parallel_loop(0, C // 16, step=1, unroll=8)
        def _loop(j):
            group(j * 16)

        @pl.when(sid == 0)
        def _():
            pltpu.sync_copy(acc_s, sh_s)
            pltpu.sync_copy(acc_r, sh_r)

        plsc.subcore_barrier()

        @pl.when(sid != 0)
        def _():
            pltpu.sync_copy(acc_s, sh_s.at[rows_v], add=True)
            pltpu.sync_copy(acc_r, sh_r.at[rows_v], add=True)

        plsc.subcore_barrier()

        @pl.when(sid == 0)
        def _():
            pltpu.sync_copy(sh_s, out_s.at[cid])
            pltpu.sync_copy(sh_r, out_r.at[cid])

    return k(v, u, batch)


def _tc_final(s_ref, r_ref, w_ref, l_ref, wq_ref, o_loss, o_lv, o_la, o_lb):
    s = s_ref[0] + s_ref[1]
    r = r_ref[0] + r_ref[1]
    wq2 = jnp.sum(wq_ref[...] ** 2)
    q = 1.0 / (1.0 + jnp.exp(-s))
    dq = q * (1.0 - q)
    vals = dq * dq * wq2 * r * w_ref[...]
    lab = l_ref[...]
    mt = lab > 1
    ma = lab == 0
    mb = lab == 1
    nt = jnp.maximum(jnp.sum(mt.astype(jnp.float32)), 1.0)
    na = jnp.maximum(jnp.sum(ma.astype(jnp.float32)), 1.0)
    nb = jnp.maximum(jnp.sum(mb.astype(jnp.float32)), 1.0)
    lv = jnp.sum(jnp.where(mt, vals, 0.0)) / nt
    la = jnp.sum(jnp.where(ma, q * q, 0.0)) / na
    lb = jnp.sum(jnp.where(mb, (q - 1.0) ** 2, 0.0)) / nb
    lgv = jnp.log(lv)
    o_loss[0, 0] = lgv + GAMMA * (la + lb)
    o_lv[0, 0] = lgv
    o_la[0, 0] = GAMMA * la
    o_lb[0, 0] = GAMMA * lb


def kernel(positions, graph_labels, weight, node_attrs, batch, atomic_masses, w_q):
    batch_i = batch.astype(jnp.int32)
    mass128 = jnp.pad(atomic_masses.astype(jnp.float32), (0, 120),
                      constant_values=1.0).reshape(1, 128)
    wq128 = jnp.pad(w_q.astype(jnp.float32), (0, 125)).reshape(1, 128)

    nodes = jax.ShapeDtypeStruct((N,), jnp.float32)
    v, u = pl.pallas_call(
        _tc_prep,
        grid=(N // BN,),
        in_specs=[
            pl.BlockSpec((3, BN), lambda g: (0, g)),
            pl.BlockSpec((8, BN), lambda g: (0, g)),
            pl.BlockSpec((1, 128), lambda g: (0, 0)),
            pl.BlockSpec((1, 128), lambda g: (0, 0)),
        ],
        out_specs=[
            pl.BlockSpec((BN,), lambda g: (g,)),
            pl.BlockSpec((BN,), lambda g: (g,)),
        ],
        out_shape=[nodes, nodes],
    )(positions.T, node_attrs.T, wq128, mass128)

    s_part, r_part = _sc_segment_sums(v, u, batch_i)

    wgt2d = weight.astype(jnp.float32).reshape(16, 128)
    lab2d = graph_labels.astype(jnp.int32).reshape(16, 128)

    scalar = jax.ShapeDtypeStruct((1, 1), jnp.float32)
    outs = pl.pallas_call(
        _tc_final,
        out_shape=[scalar, scalar, scalar, scalar],
        out_specs=[pl.BlockSpec(memory_space=pltpu.SMEM)] * 4,
    )(s_part, r_part, wgt2d, lab2d, wq128)
    loss, lgv, la, lb = (o.reshape(()) for o in outs)
    return (loss, lgv, la, lb)


# speedup vs baseline: 55.8672x; 1.0755x over previous
import functools

import jax
import jax.numpy as jnp
from jax import lax
from jax.experimental import pallas as pl
from jax.experimental.pallas import tpu as pltpu
from jax.experimental.pallas import tpu_sc as plsc

N = 131072
G = 2048
NW = 32
C = N // NW
BN = 65536
GAMMA = 10000.0


def _tc_prep(pos_ref, attr_ref, wq_ref, mass_ref, v_ref, u_ref):
    dn = (((1,), (0,)), ((), ()))
    wq3 = wq_ref[:, :3]
    im8 = 1.0 / mass_ref[:, :8]
    v_ref[...] = jax.lax.dot_general(
        wq3, pos_ref[...], dn, preferred_element_type=jnp.float32).reshape(BN)
    u_ref[...] = jax.lax.dot_general(
        im8, attr_ref[...], dn, preferred_element_type=jnp.float32).reshape(BN)


def _sc_segment_sums(v, u, batch):
    mesh = plsc.VectorSubcoreMesh(core_axis_name="c", subcore_axis_name="s")

    @functools.partial(
        pl.kernel,
        out_type=[
            jax.ShapeDtypeStruct((NW, 16, 128), jnp.float32),
            jax.ShapeDtypeStruct((NW, 16, 128), jnp.float32),
        ],
        mesh=mesh,
        compiler_params=pltpu.CompilerParams(needs_layout_passes=False),
        scratch_types=[
            pltpu.VMEM((C,), jnp.float32),
            pltpu.VMEM((C,), jnp.float32),
            pltpu.VMEM((C + 16,), jnp.int32),
            pltpu.VMEM((16, 128), jnp.float32),
            pltpu.VMEM((16, 128), jnp.float32),
            pltpu.SemaphoreType.DMA,
        ],
    )
    def k(v_hbm, u_hbm, batch_hbm, out_s, out_r,
          v_v, u_v, batch_v, acc_s, acc_r, sem):
        cid = lax.axis_index("c")
        sid = lax.axis_index("s")
        wid = sid * 2 + cid
        base = wid * C

        cp1 = pltpu.async_copy(v_hbm.at[pl.ds(base, C)], v_v, sem)
        cp2 = pltpu.async_copy(u_hbm.at[pl.ds(base, C)], u_v, sem)
        cp3 = pltpu.async_copy(batch_hbm.at[pl.ds(base, C)], batch_v.at[pl.ds(0, C)], sem)

        lane = lax.iota(jnp.int32, 16)
        zeros16 = jnp.zeros((16,), jnp.float32)
        for rr in range(16):
            for cc in range(8):
                acc_s[rr, pl.ds(cc * 16, 16)] = zeros16
                acc_r[rr, pl.ds(cc * 16, 16)] = zeros16

        cp1.wait()
        cp2.wait()
        cp3.wait()

        batch_v[pl.ds(C, 16)] = plsc.load_gather(
            batch_v, [jnp.full((16,), C - 1, jnp.int32)])

        is15 = lane == 15

        def group(b16):
            idx = batch_v[pl.ds(b16, 16)]
            nxt = plsc.load_gather(batch_v, [lane + (b16 + 1)])
            cv = plsc.cumsum(v_v[pl.ds(b16, 16)])
            cu = plsc.cumsum(u_v[pl.ds(b16, 16)])
            bnd = idx != nxt
            mend = bnd | is15
            mneg = bnd & jnp.logical_not(is15)
            ri = idx >> 7
            ci = idx & 127
            rn = nxt >> 7
            cn = nxt & 127
            plsc.addupdate_scatter(acc_s, [ri, ci], cv, mask=mend)
            plsc.addupdate_scatter(acc_s, [rn, cn], -cv, mask=mneg)
            plsc.addupdate_scatter(acc_r, [ri, ci], cu, mask=mend)
            plsc.addupdate_scatter(acc_r, [rn, cn], -cu, mask=mneg)

        @plsc.parallel_loop(0, C // 16, step=1, unroll=8)
        def _loop(j):
            group(j * 16)

        pltpu.sync_copy(acc_s, out_s.at[wid])
        pltpu.sync_copy(acc_r, out_r.at[wid])

    return k(v, u, batch)


def _tc_final(s_ref, r_ref, w_ref, l_ref, wq_ref, o_loss, o_lv, o_la, o_lb):
    s = jnp.sum(s_ref[...], axis=0)
    r = jnp.sum(r_ref[...], axis=0)
    wq2 = jnp.sum(wq_ref[...] ** 2)
    q = 1.0 / (1.0 + jnp.exp(-s))
    dq = q * (1.0 - q)
    vals = dq * dq * wq2 * r * w_ref[...]
    lab = l_ref[...]
    mt = lab > 1
    ma = lab == 0
    mb = lab == 1
    nt = jnp.maximum(jnp.sum(mt.astype(jnp.float32)), 1.0)
    na = jnp.maximum(jnp.sum(ma.astype(jnp.float32)), 1.0)
    nb = jnp.maximum(jnp.sum(mb.astype(jnp.float32)), 1.0)
    lv = jnp.sum(jnp.where(mt, vals, 0.0)) / nt
    la = jnp.sum(jnp.where(ma, q * q, 0.0)) / na
    lb = jnp.sum(jnp.where(mb, (q - 1.0) ** 2, 0.0)) / nb
    lgv = jnp.log(lv)
    o_loss[0, 0] = lgv + GAMMA * (la + lb)
    o_lv[0, 0] = lgv
    o_la[0, 0] = GAMMA * la
    o_lb[0, 0] = GAMMA * lb


def kernel(positions, graph_labels, weight, node_attrs, batch, atomic_masses, w_q):
    batch_i = batch.astype(jnp.int32)
    mass128 = jnp.pad(atomic_masses.astype(jnp.float32), (0, 120),
                      constant_values=1.0).reshape(1, 128)
    wq128 = jnp.pad(w_q.astype(jnp.float32), (0, 125)).reshape(1, 128)

    nodes = jax.ShapeDtypeStruct((N,), jnp.float32)
    v, u = pl.pallas_call(
        _tc_prep,
        grid=(N // BN,),
        in_specs=[
            pl.BlockSpec((3, BN), lambda g: (0, g)),
            pl.BlockSpec((8, BN), lambda g: (0, g)),
            pl.BlockSpec((1, 128), lambda g: (0, 0)),
            pl.BlockSpec((1, 128), lambda g: (0, 0)),
        ],
        out_specs=[
            pl.BlockSpec((BN,), lambda g: (g,)),
            pl.BlockSpec((BN,), lambda g: (g,)),
        ],
        out_shape=[nodes, nodes],
    )(positions.T, node_attrs.T, wq128, mass128)

    s_part, r_part = _sc_segment_sums(v, u, batch_i)

    wgt2d = weight.astype(jnp.float32).reshape(16, 128)
    lab2d = graph_labels.astype(jnp.int32).reshape(16, 128)

    scalar = jax.ShapeDtypeStruct((1, 1), jnp.float32)
    outs = pl.pallas_call(
        _tc_final,
        out_shape=[scalar, scalar, scalar, scalar],
        out_specs=[pl.BlockSpec(memory_space=pltpu.SMEM)] * 4,
    )(s_part, r_part, wgt2d, lab2d, wq128)
    loss, lgv, la, lb = (o.reshape(()) for o in outs)
    return (loss, lgv, la, lb)


# speedup vs baseline: 55.9365x; 1.0012x over previous
import functools

import jax
import jax.numpy as jnp
from jax import lax
from jax.experimental import pallas as pl
from jax.experimental.pallas import tpu as pltpu
from jax.experimental.pallas import tpu_sc as plsc

N = 131072
G = 2048
NW = 32
C = N // NW
BN = 65536
GAMMA = 10000.0


def _tc_prep(pos_ref, attr_ref, wq_ref, mass_ref, v_ref, u_ref):
    dn = (((1,), (0,)), ((), ()))
    wq3 = wq_ref[:, :3]
    im8 = 1.0 / mass_ref[:, :8]
    v_ref[...] = jax.lax.dot_general(
        wq3, pos_ref[...], dn, preferred_element_type=jnp.float32).reshape(BN)
    u_ref[...] = jax.lax.dot_general(
        im8, attr_ref[...], dn, preferred_element_type=jnp.float32).reshape(BN)


def _sc_segment_sums(v, u, batch):
    mesh = plsc.VectorSubcoreMesh(core_axis_name="c", subcore_axis_name="s")

    @functools.partial(
        pl.kernel,
        out_type=[
            jax.ShapeDtypeStruct((NW, 16, 128), jnp.float32),
            jax.ShapeDtypeStruct((NW, 16, 128), jnp.float32),
        ],
        mesh=mesh,
        compiler_params=pltpu.CompilerParams(needs_layout_passes=False),
        scratch_types=[
            pltpu.VMEM((C,), jnp.float32),
            pltpu.VMEM((C,), jnp.float32),
            pltpu.VMEM((C + 16,), jnp.int32),
            pltpu.VMEM((16, 128), jnp.float32),
            pltpu.VMEM((16, 128), jnp.float32),
            pltpu.SemaphoreType.DMA,
        ],
    )
    def k(v_hbm, u_hbm, batch_hbm, out_s, out_r,
          v_v, u_v, batch_v, acc_s, acc_r, sem):
        cid = lax.axis_index("c")
        sid = lax.axis_index("s")
        wid = sid * 2 + cid
        base = wid * C

        cp1 = pltpu.async_copy(v_hbm.at[pl.ds(base, C)], v_v, sem)
        cp2 = pltpu.async_copy(u_hbm.at[pl.ds(base, C)], u_v, sem)
        cp3 = pltpu.async_copy(batch_hbm.at[pl.ds(base, C)], batch_v.at[pl.ds(0, C)], sem)

        lane = lax.iota(jnp.int32, 16)
        zeros16 = jnp.zeros((16,), jnp.float32)
        for rr in range(16):
            for cc in range(8):
                acc_s[rr, pl.ds(cc * 16, 16)] = zeros16
                acc_r[rr, pl.ds(cc * 16, 16)] = zeros16

        cp1.wait()
        cp2.wait()
        cp3.wait()

        batch_v[pl.ds(C, 16)] = plsc.load_gather(
            batch_v, [jnp.full((16,), C - 1, jnp.int32)])

        is15 = lane == 15

        def group(b16):
            idx = batch_v[pl.ds(b16, 16)]
            nxt = plsc.load_gather(batch_v, [lane + (b16 + 1)])
            cv = plsc.cumsum(v_v[pl.ds(b16, 16)])
            cu = plsc.cumsum(u_v[pl.ds(b16, 16)])
            bnd = idx != nxt
            mend = bnd | is15
            mneg = bnd & jnp.logical_not(is15)
            ri = idx >> 7
            ci = idx & 127
            rn = nxt >> 7
            cn = nxt & 127
            plsc.addupdate_scatter(acc_s, [ri, ci], cv, mask=mend)
            plsc.addupdate_scatter(acc_s, [rn, cn], -cv, mask=mneg)
            plsc.addupdate_scatter(acc_r, [ri, ci], cu, mask=mend)
            plsc.addupdate_scatter(acc_r, [rn, cn], -cu, mask=mneg)

        @plsc.parallel_loop(0, C // 16, step=1, unroll=8)
        def _loop(j):
            group(j * 16)

        cps = pltpu.async_copy(acc_s, out_s.at[wid], sem)
        cpr = pltpu.async_copy(acc_r, out_r.at[wid], sem)
        cps.wait()
        cpr.wait()

    return k(v, u, batch)


def _tc_final(s_ref, r_ref, w_ref, l_ref, wq_ref, o_loss, o_lv, o_la, o_lb):
    s = jnp.sum(s_ref[...], axis=0)
    r = jnp.sum(r_ref[...], axis=0)
    wq2 = jnp.sum(wq_ref[...] ** 2)
    q = 1.0 / (1.0 + jnp.exp(-s))
    dq = q * (1.0 - q)
    vals = dq * dq * wq2 * r * w_ref[...]
    lab = l_ref[...]
    mt = lab > 1
    ma = lab == 0
    mb = lab == 1
    nt = jnp.maximum(jnp.sum(mt.astype(jnp.float32)), 1.0)
    na = jnp.maximum(jnp.sum(ma.astype(jnp.float32)), 1.0)
    nb = jnp.maximum(jnp.sum(mb.astype(jnp.float32)), 1.0)
    lv = jnp.sum(jnp.where(mt, vals, 0.0)) / nt
    la = jnp.sum(jnp.where(ma, q * q, 0.0)) / na
    lb = jnp.sum(jnp.where(mb, (q - 1.0) ** 2, 0.0)) / nb
    lgv = jnp.log(lv)
    o_loss[0, 0] = lgv + GAMMA * (la + lb)
    o_lv[0, 0] = lgv
    o_la[0, 0] = GAMMA * la
    o_lb[0, 0] = GAMMA * lb


def kernel(positions, graph_labels, weight, node_attrs, batch, atomic_masses, w_q):
    batch_i = batch.astype(jnp.int32)
    mass128 = jnp.pad(atomic_masses.astype(jnp.float32), (0, 120),
                      constant_values=1.0).reshape(1, 128)
    wq128 = jnp.pad(w_q.astype(jnp.float32), (0, 125)).reshape(1, 128)

    nodes = jax.ShapeDtypeStruct((N,), jnp.float32)
    v, u = pl.pallas_call(
        _tc_prep,
        grid=(N // BN,),
        in_specs=[
            pl.BlockSpec((3, BN), lambda g: (0, g)),
            pl.BlockSpec((8, BN), lambda g: (0, g)),
            pl.BlockSpec((1, 128), lambda g: (0, 0)),
            pl.BlockSpec((1, 128), lambda g: (0, 0)),
        ],
        out_specs=[
            pl.BlockSpec((BN,), lambda g: (g,)),
            pl.BlockSpec((BN,), lambda g: (g,)),
        ],
        out_shape=[nodes, nodes],
    )(positions.T, node_attrs.T, wq128, mass128)

    s_part, r_part = _sc_segment_sums(v, u, batch_i)

    wgt2d = weight.astype(jnp.float32).reshape(16, 128)
    lab2d = graph_labels.astype(jnp.int32).reshape(16, 128)

    scalar = jax.ShapeDtypeStruct((1, 1), jnp.float32)
    outs = pl.pallas_call(
        _tc_final,
        out_shape=[scalar, scalar, scalar, scalar],
        out_specs=[pl.BlockSpec(memory_space=pltpu.SMEM)] * 4,
    )(s_part, r_part, wgt2d, lab2d, wq128)
    loss, lgv, la, lb = (o.reshape(()) for o in outs)
    return (loss, lgv, la, lb)
